# baseline (speedup 1.0000x reference)
import functools

import jax
import jax.numpy as jnp
from jax import lax
from jax.experimental import pallas as pl
from jax.experimental.pallas import tpu as pltpu
from jax.experimental.pallas import tpu_sc as plsc

K = 32
B = 16384
NC = 2
NS = 16
NW = NC * NS
BPW = B // NW
CHUNK = 128
NCHUNK = BPW // CHUNK


def _bpr_mesh_kernel():
    mesh = plsc.VectorSubcoreMesh(core_axis_name="c", subcore_axis_name="s")

    @functools.partial(
        pl.kernel,
        mesh=mesh,
        compiler_params=pltpu.CompilerParams(use_tc_tiling_on_sc=False),
        out_type=[
            jax.ShapeDtypeStruct((B,), jnp.float32),
            jax.ShapeDtypeStruct((B,), jnp.float32),
        ],
        scratch_types=[
            pltpu.VMEM((NCHUNK, CHUNK), jnp.int32),
            pltpu.VMEM((NCHUNK, CHUNK), jnp.int32),
            pltpu.VMEM((NCHUNK, CHUNK), jnp.int32),
            pltpu.VMEM((BPW, K), jnp.float32),
            pltpu.VMEM((BPW, K), jnp.float32),
            pltpu.VMEM((BPW, K), jnp.float32),
            pltpu.VMEM((BPW,), jnp.float32),
            pltpu.VMEM((BPW,), jnp.float32),
            pltpu.VMEM((BPW,), jnp.float32),
            pltpu.VMEM((BPW,), jnp.float32),
            pltpu.VMEM((BPW,), jnp.float32),
            pltpu.SemaphoreType.DMA,
        ],
    )
    def k(u_hbm, i_hbm, j_hbm, gu_hbm, gi_hbm, bu_hbm, bi_hbm,
          xui_hbm, xuj_hbm,
          u_v, i_v, j_v, lu, li, lj, bu_v, bi_v, bj_v, xui_v, xuj_v, sem):
        wid = lax.axis_index("s") * NC + lax.axis_index("c")
        base = wid * BPW

        pltpu.sync_copy(u_hbm.at[wid], u_v)
        pltpu.sync_copy(i_hbm.at[wid], i_v)
        pltpu.sync_copy(j_hbm.at[wid], j_v)

        copies = []
        for c in range(NCHUNK):
            sl = pl.ds(c * CHUNK, CHUNK)
            copies.append(pltpu.async_copy(gu_hbm.at[u_v.at[c]], lu.at[sl], sem))
            copies.append(pltpu.async_copy(gi_hbm.at[i_v.at[c]], li.at[sl], sem))
            copies.append(pltpu.async_copy(gi_hbm.at[j_v.at[c]], lj.at[sl], sem))
            copies.append(pltpu.async_copy(bu_hbm.at[u_v.at[c]], bu_v.at[sl], sem))
            copies.append(pltpu.async_copy(bi_hbm.at[i_v.at[c]], bi_v.at[sl], sem))
            copies.append(pltpu.async_copy(bi_hbm.at[j_v.at[c]], bj_v.at[sl], sem))
        for cp in copies:
            cp.wait()

        rows16 = lax.iota(jnp.int32, 16)
        perms = [jnp.bitwise_xor(rows16, k) for k in (8, 4, 2, 1)]
        zeros = jnp.zeros((16,), jnp.float32)
        gd = lax.GatherDimensionNumbers(
            offset_dims=(), collapsed_slice_dims=(0,), start_index_map=(0,))

        def take16(v, idx):
            return lax.gather(
                v, idx[:, None], gd, slice_sizes=(1,),
                mode=lax.GatherScatterMode.PROMISE_IN_BOUNDS)

        def g_body(g, carry):
            r = g * 16
            acc_ui = bu_v[pl.ds(r, 16)] + bi_v[pl.ds(r, 16)]
            acc_uj = bu_v[pl.ds(r, 16)] + bj_v[pl.ds(r, 16)]
            for rr in range(16):
                lu0 = lu[r + rr, pl.ds(0, 16)]
                lu1 = lu[r + rr, pl.ds(16, 16)]
                li0 = li[r + rr, pl.ds(0, 16)]
                li1 = li[r + rr, pl.ds(16, 16)]
                lj0 = lj[r + rr, pl.ds(0, 16)]
                lj1 = lj[r + rr, pl.ds(16, 16)]
                t_ui = lu0 * li0 + lu1 * li1
                t_uj = lu0 * lj0 + lu1 * lj1
                for p in perms:
                    t_ui = t_ui + take16(t_ui, p)
                    t_uj = t_uj + take16(t_uj, p)
                m = rows16 == rr
                acc_ui = acc_ui + jnp.where(m, t_ui, zeros)
                acc_uj = acc_uj + jnp.where(m, t_uj, zeros)
            xui_v[pl.ds(r, 16)] = acc_ui
            xuj_v[pl.ds(r, 16)] = acc_uj
            return carry

        lax.fori_loop(0, BPW // 16, g_body, 0)

        pltpu.sync_copy(xui_v, xui_hbm.at[pl.ds(base, BPW)])
        pltpu.sync_copy(xuj_v, xuj_hbm.at[pl.ds(base, BPW)])

    return k


_bpr_sc = _bpr_mesh_kernel()


def _loss_body(xui_ref, xuj_ref, out_ref):
    d = xui_ref[...] - xuj_ref[...]
    ls = jnp.minimum(d, 0.0) - jnp.log(1.0 + jnp.exp(-jnp.abs(d)))
    out_ref[...] = jnp.broadcast_to(-jnp.sum(ls) / B, (1, 1))


def _loss_tc(x_ui, x_uj):
    out = pl.pallas_call(
        _loss_body,
        out_shape=jax.ShapeDtypeStruct((1, 1), jnp.float32),
    )(x_ui.reshape(128, 128), x_uj.reshape(128, 128))
    return out[0, 0]


@jax.jit
def kernel(u, i, j, gamma_u, gamma_i, beta_u, beta_i):
    u_r = u.astype(jnp.int32).reshape(NW, NCHUNK, CHUNK)
    i_r = i.astype(jnp.int32).reshape(NW, NCHUNK, CHUNK)
    j_r = j.astype(jnp.int32).reshape(NW, NCHUNK, CHUNK)
    x_ui, x_uj = _bpr_sc(u_r, i_r, j_r, gamma_u, gamma_i,
                         beta_u.reshape(-1), beta_i.reshape(-1))
    loss = _loss_tc(x_ui, x_uj)
    return (x_ui, x_uj, loss)


# speedup vs baseline: 2.8165x; 2.8165x over previous
import functools

import jax
import jax.numpy as jnp
from jax import lax
from jax.experimental import pallas as pl
from jax.experimental.pallas import tpu as pltpu
from jax.experimental.pallas import tpu_sc as plsc

K = 32
B = 16384
N = 1000000
NC = 2
NS = 16
NW = NC * NS
BPW = B // NW
CHUNK = 128
NCHUNK = BPW // CHUNK

CB = 1024
NFULL = N // CB
EXTRA_OFF = NFULL * CB
EXTRA = 512
T0 = EXTRA_OFF + EXTRA
NTAIL = N - T0


def _stage_kernel():
    mesh = plsc.VectorSubcoreMesh(core_axis_name="c", subcore_axis_name="s")

    @functools.partial(
        pl.kernel,
        mesh=mesh,
        out_type=[
            jax.ShapeDtypeStruct((K * N,), jnp.float32),
            jax.ShapeDtypeStruct((K * N,), jnp.float32),
            jax.ShapeDtypeStruct((N,), jnp.float32),
            jax.ShapeDtypeStruct((N,), jnp.float32),
        ],
        scratch_types=[
            pltpu.VMEM((K * CB,), jnp.float32),
            pltpu.VMEM((K * CB,), jnp.float32),
            pltpu.VMEM((CB,), jnp.float32),
            pltpu.VMEM((CB,), jnp.float32),
            pltpu.SemaphoreType.DMA,
        ],
    )
    def k(guT_hbm, giT_hbm, buT_hbm, biT_hbm,
          su_hbm, si_hbm, sbu_hbm, sbi_hbm,
          bu, bi, bbu, bbi, sem):
        wid = lax.axis_index("s") * NC + lax.axis_index("c")
        nblk = NFULL // NW + 1

        def stage_block(off, width):
            reads = [
                pltpu.async_copy(buT_hbm.at[0, pl.ds(off, width)],
                                 bbu.at[pl.ds(0, width)], sem),
                pltpu.async_copy(biT_hbm.at[0, pl.ds(off, width)],
                                 bbi.at[pl.ds(0, width)], sem),
            ]
            for c in range(K):
                csl = pl.ds(c * CB, width)
                reads.append(pltpu.async_copy(
                    guT_hbm.at[c, pl.ds(off, width)], bu.at[csl], sem))
                reads.append(pltpu.async_copy(
                    giT_hbm.at[c, pl.ds(off, width)], bi.at[csl], sem))
            for cp in reads:
                cp.wait()
            writes = [
                pltpu.async_copy(bbu.at[pl.ds(0, width)],
                                 sbu_hbm.at[pl.ds(off, width)], sem),
                pltpu.async_copy(bbi.at[pl.ds(0, width)],
                                 sbi_hbm.at[pl.ds(off, width)], sem),
            ]
            for c in range(K):
                csl = pl.ds(c * CB, width)
                writes.append(pltpu.async_copy(
                    bu.at[csl], su_hbm.at[pl.ds(c * N + off, width)], sem))
                writes.append(pltpu.async_copy(
                    bi.at[csl], si_hbm.at[pl.ds(c * N + off, width)], sem))
            for cp in writes:
                cp.wait()

        def body(n, carry):
            blk = wid + n * NW

            @pl.when(blk < NFULL)
            def _():
                stage_block(pl.multiple_of(blk * CB, CB), CB)
            return carry

        lax.fori_loop(0, nblk, body, 0)

        @pl.when(wid == 0)
        def _():
            stage_block(EXTRA_OFF, EXTRA)

    return k


def _gather_kernel():
    mesh = plsc.VectorSubcoreMesh(core_axis_name="c", subcore_axis_name="s")

    @functools.partial(
        pl.kernel,
        mesh=mesh,
        compiler_params=pltpu.CompilerParams(use_tc_tiling_on_sc=False),
        out_type=[
            jax.ShapeDtypeStruct((K * B,), jnp.float32),
            jax.ShapeDtypeStruct((K * B,), jnp.float32),
            jax.ShapeDtypeStruct((K * B,), jnp.float32),
            jax.ShapeDtypeStruct((B,), jnp.float32),
            jax.ShapeDtypeStruct((B,), jnp.float32),
            jax.ShapeDtypeStruct((B,), jnp.float32),
        ],
        scratch_types=[
            pltpu.VMEM((NCHUNK, CHUNK), jnp.int32),
            pltpu.VMEM((NCHUNK, CHUNK), jnp.int32),
            pltpu.VMEM((NCHUNK, CHUNK), jnp.int32),
            pltpu.VMEM((K, BPW), jnp.float32),
            pltpu.VMEM((K, BPW), jnp.float32),
            pltpu.VMEM((K, BPW), jnp.float32),
            pltpu.VMEM((BPW,), jnp.float32),
            pltpu.VMEM((BPW,), jnp.float32),
            pltpu.VMEM((BPW,), jnp.float32),
            pltpu.VMEM((3 * NCHUNK * CHUNK,), jnp.float32),
            pltpu.SemaphoreType.DMA,
        ],
    )
    def k(u_hbm, i_hbm, j_hbm, su_hbm, si_hbm, sbu_hbm, sbi_hbm,
          lu_hbm, li_hbm, lj_hbm, obu_hbm, obi_hbm, obj_hbm,
          u_v, i_v, j_v, vu, vi, vj, bu_v, bi_v, bj_v, drain_v, sem):
        wid = lax.axis_index("s") * NC + lax.axis_index("c")
        base = wid * BPW

        pltpu.sync_copy(u_hbm.at[wid], u_v)
        pltpu.sync_copy(i_hbm.at[wid], i_v)
        pltpu.sync_copy(j_hbm.at[wid], j_v)

        bias_copies = []
        for t in range(NCHUNK):
            sl = pl.ds(t * CHUNK, CHUNK)
            bias_copies.append(pltpu.async_copy(
                sbu_hbm.at[u_v.at[t]], bu_v.at[sl], sem))
            bias_copies.append(pltpu.async_copy(
                sbi_hbm.at[i_v.at[t]], bi_v.at[sl], sem))
            bias_copies.append(pltpu.async_copy(
                sbi_hbm.at[j_v.at[t]], bj_v.at[sl], sem))

        def fire(c, carry):
            for t in range(NCHUNK):
                sl = pl.ds(t * CHUNK, CHUNK)
                pltpu.async_copy(su_hbm.at[c].at[u_v.at[t]], vu.at[c, sl], sem)
                pltpu.async_copy(si_hbm.at[c].at[i_v.at[t]], vi.at[c, sl], sem)
                pltpu.async_copy(si_hbm.at[c].at[j_v.at[t]], vj.at[c, sl], sem)

            @pl.when(c > 0)
            def _():
                pltpu.make_async_copy(
                    sbu_hbm.at[pl.ds(0, 3 * NCHUNK * CHUNK)], drain_v, sem).wait()
            return carry

        lax.fori_loop(0, K, fire, 0)
        pltpu.make_async_copy(
            sbu_hbm.at[pl.ds(0, 3 * NCHUNK * CHUNK)], drain_v, sem).wait()
        for cp in bias_copies:
            cp.wait()

        out_copies = [
            pltpu.async_copy(bu_v, obu_hbm.at[pl.ds(base, BPW)], sem),
            pltpu.async_copy(bi_v, obi_hbm.at[pl.ds(base, BPW)], sem),
            pltpu.async_copy(bj_v, obj_hbm.at[pl.ds(base, BPW)], sem),
        ]
        for c in range(K):
            out_copies.append(pltpu.async_copy(
                vu.at[c], lu_hbm.at[pl.ds(c * B + base, BPW)], sem))
            out_copies.append(pltpu.async_copy(
                vi.at[c], li_hbm.at[pl.ds(c * B + base, BPW)], sem))
            out_copies.append(pltpu.async_copy(
                vj.at[c], lj_hbm.at[pl.ds(c * B + base, BPW)], sem))
        for cp in out_copies:
            cp.wait()

    return k


_bpr_stage = _stage_kernel()
_bpr_gather = _gather_kernel()


def _epilogue_body(u_ref, i_ref, j_ref, lu_ref, li_ref, lj_ref,
                   bu_ref, bi_ref, bj_ref,
                   tgu_ref, tgi_ref, tbu_ref, tbi_ref,
                   xui_ref, xuj_ref, loss_ref):
    u = u_ref[...]
    i = i_ref[...]
    j = j_ref[...]
    tail_ids = lax.broadcasted_iota(jnp.int32, (NTAIL, B), 0) + T0

    def fix(latent, bias_vals, idx, tg, tb):
        onehot = (tail_ids == idx).astype(jnp.float32)
        is_tail = idx >= T0
        lat_fix = jnp.dot(tg, onehot,
                          preferred_element_type=jnp.float32)
        b_fix = jnp.dot(tb, onehot,
                        preferred_element_type=jnp.float32)
        latent = jnp.where(is_tail, lat_fix, latent)
        bias_vals = jnp.where(is_tail, b_fix, bias_vals)
        return latent, bias_vals

    lu, bu = fix(lu_ref[...], bu_ref[...], u, tgu_ref[...], tbu_ref[...])
    li, bi = fix(li_ref[...], bi_ref[...], i, tgi_ref[...], tbi_ref[...])
    lj, bj = fix(lj_ref[...], bj_ref[...], j, tgi_ref[...], tbi_ref[...])

    x_ui = jnp.sum(lu * li, axis=0, keepdims=True) + bu + bi
    x_uj = jnp.sum(lu * lj, axis=0, keepdims=True) + bu + bj
    xui_ref[...] = x_ui
    xuj_ref[...] = x_uj
    d = x_ui - x_uj
    ls = jnp.minimum(d, 0.0) - jnp.log(1.0 + jnp.exp(-jnp.abs(d)))
    loss_ref[...] = jnp.broadcast_to(-jnp.sum(ls) / B, (1, 1))


def _epilogue_tc(u, i, j, lu, li, lj, bu, bi, bj, tgu, tgi, tbu, tbi):
    return pl.pallas_call(
        _epilogue_body,
        out_shape=[
            jax.ShapeDtypeStruct((1, B), jnp.float32),
            jax.ShapeDtypeStruct((1, B), jnp.float32),
            jax.ShapeDtypeStruct((1, 1), jnp.float32),
        ],
    )(u.reshape(1, B), i.reshape(1, B), j.reshape(1, B),
      lu.reshape(K, B), li.reshape(K, B), lj.reshape(K, B),
      bu.reshape(1, B), bi.reshape(1, B), bj.reshape(1, B),
      tgu, tgi, tbu, tbi)


@jax.jit
def kernel(u, i, j, gamma_u, gamma_i, beta_u, beta_i):
    u32 = u.astype(jnp.int32)
    i32 = i.astype(jnp.int32)
    j32 = j.astype(jnp.int32)
    u_r = u32.reshape(NW, NCHUNK, CHUNK)
    i_r = i32.reshape(NW, NCHUNK, CHUNK)
    j_r = j32.reshape(NW, NCHUNK, CHUNK)
    su, si, sbu, sbi = _bpr_stage(gamma_u.T, gamma_i.T, beta_u.T, beta_i.T)
    lu, li, lj, bu, bi, bj = _bpr_gather(u_r, i_r, j_r,
                                         su.reshape(K, N), si.reshape(K, N),
                                         sbu, sbi)
    tgu = gamma_u[T0:].T
    tgi = gamma_i[T0:].T
    tbu = beta_u[T0:].reshape(1, NTAIL)
    tbi = beta_i[T0:].reshape(1, NTAIL)
    x_ui, x_uj, loss = _epilogue_tc(u32, i32, j32, lu, li, lj, bu, bi, bj,
                                    tgu, tgi, tbu, tbi)
    return (x_ui.reshape(B), x_uj.reshape(B), loss[0, 0])


# speedup vs baseline: 3.0467x; 1.0817x over previous
import functools

import jax
import jax.numpy as jnp
from jax import lax
from jax.experimental import pallas as pl
from jax.experimental.pallas import tpu as pltpu
from jax.experimental.pallas import tpu_sc as plsc

K = 32
B = 16384
N = 1000000
NC = 2
NS = 16
NW = NC * NS
BPW = B // NW
CHUNK = 128
NCHUNK = BPW // CHUNK

CB = 512
T0 = (N // 128) * 128
NFULL = T0 // CB
NTAIL = N - T0


def _stage_kernel():
    mesh = plsc.VectorSubcoreMesh(core_axis_name="c", subcore_axis_name="s")

    @functools.partial(
        pl.kernel,
        mesh=mesh,
        out_type=[
            jax.ShapeDtypeStruct((K * N,), jnp.float32),
            jax.ShapeDtypeStruct((K * N,), jnp.float32),
            jax.ShapeDtypeStruct((N,), jnp.float32),
            jax.ShapeDtypeStruct((N,), jnp.float32),
        ],
        scratch_types=[
            pltpu.VMEM((3 * K * CB,), jnp.float32),
            pltpu.VMEM((3 * K * CB,), jnp.float32),
            pltpu.VMEM((3 * CB,), jnp.float32),
            pltpu.VMEM((3 * CB,), jnp.float32),
            pltpu.SemaphoreType.DMA,
            pltpu.SemaphoreType.DMA,
        ],
    )
    def k(guT_hbm, giT_hbm, buT_hbm, biT_hbm,
          su_hbm, si_hbm, sbu_hbm, sbi_hbm,
          bu, bi, bbu, bbi, sem_r, sem_w):
        wid = lax.axis_index("s") * NC + lax.axis_index("c")
        nblk = NFULL // NW + 1

        def fire_reads(par, off):
            pltpu.async_copy(buT_hbm.at[0, pl.ds(off, CB)], bbu.at[pl.ds(par * CB, CB)], sem_r)
            pltpu.async_copy(biT_hbm.at[0, pl.ds(off, CB)], bbi.at[pl.ds(par * CB, CB)], sem_r)
            for c in range(K):
                csl = pl.ds(c * CB, CB)
                pltpu.async_copy(
                    guT_hbm.at[c, pl.ds(off, CB)], bu.at[pl.ds(par * K * CB + c * CB, CB)], sem_r)
                pltpu.async_copy(
                    giT_hbm.at[c, pl.ds(off, CB)], bi.at[pl.ds(par * K * CB + c * CB, CB)], sem_r)

        def fire_writes(par, off):
            pltpu.async_copy(bbu.at[pl.ds(par * CB, CB)], sbu_hbm.at[pl.ds(off, CB)], sem_w)
            pltpu.async_copy(bbi.at[pl.ds(par * CB, CB)], sbi_hbm.at[pl.ds(off, CB)], sem_w)
            for c in range(K):
                csl = pl.ds(c * CB, CB)
                pltpu.async_copy(
                    bu.at[pl.ds(par * K * CB + c * CB, CB)], su_hbm.at[pl.ds(c * N + off, CB)], sem_w)
                pltpu.async_copy(
                    bi.at[pl.ds(par * K * CB + c * CB, CB)], si_hbm.at[pl.ds(c * N + off, CB)], sem_w)

        def drain(sem):
            pltpu.make_async_copy(
                buT_hbm.at[0, pl.ds(0, CB)], bbu.at[pl.ds(0, CB)], sem).wait()
            pltpu.make_async_copy(
                buT_hbm.at[0, pl.ds(0, CB)], bbi.at[pl.ds(0, CB)], sem).wait()
            for c in range(K):
                csl = pl.ds(c * CB, CB)
                pltpu.make_async_copy(
                    guT_hbm.at[0, pl.ds(0, CB)], bu.at[csl], sem).wait()
                pltpu.make_async_copy(
                    guT_hbm.at[0, pl.ds(0, CB)], bi.at[csl], sem).wait()

        def blk_off(n):
            return pl.multiple_of((wid + n * NW) * CB, CB)

        fire_reads(0, blk_off(0))

        def body(n, carry):
            blk = wid + n * NW

            @pl.when(blk < NFULL)
            def _():
                @pl.when(n > 1)
                def _():
                    drain(sem_w)

                @pl.when(wid + (n + 1) * NW < NFULL)
                def _():
                    for par in (0, 1, 2):
                        @pl.when((n + 1) % 3 == par)
                        def _():
                            fire_reads(par, blk_off(n + 1))

                drain(sem_r)
                for par in (0, 1, 2):
                    @pl.when(n % 3 == par)
                    def _():
                        fire_writes(par, blk_off(n))
            return carry

        lax.fori_loop(0, nblk, body, 0)
        drain(sem_w)
        drain(sem_w)

    return k


def _gather_kernel():
    mesh = plsc.VectorSubcoreMesh(core_axis_name="c", subcore_axis_name="s")

    @functools.partial(
        pl.kernel,
        mesh=mesh,
        compiler_params=pltpu.CompilerParams(use_tc_tiling_on_sc=False),
        out_type=[
            jax.ShapeDtypeStruct((K * B,), jnp.float32),
            jax.ShapeDtypeStruct((K * B,), jnp.float32),
            jax.ShapeDtypeStruct((K * B,), jnp.float32),
            jax.ShapeDtypeStruct((B,), jnp.float32),
            jax.ShapeDtypeStruct((B,), jnp.float32),
            jax.ShapeDtypeStruct((B,), jnp.float32),
        ],
        scratch_types=[
            pltpu.VMEM((NCHUNK, CHUNK), jnp.int32),
            pltpu.VMEM((NCHUNK, CHUNK), jnp.int32),
            pltpu.VMEM((NCHUNK, CHUNK), jnp.int32),
            pltpu.VMEM((K, BPW), jnp.float32),
            pltpu.VMEM((K, BPW), jnp.float32),
            pltpu.VMEM((K, BPW), jnp.float32),
            pltpu.VMEM((BPW,), jnp.float32),
            pltpu.VMEM((BPW,), jnp.float32),
            pltpu.VMEM((BPW,), jnp.float32),
            pltpu.VMEM((3 * NCHUNK * CHUNK,), jnp.float32),
            pltpu.SemaphoreType.DMA,
        ],
    )
    def k(u_hbm, i_hbm, j_hbm, su_hbm, si_hbm, sbu_hbm, sbi_hbm,
          lu_hbm, li_hbm, lj_hbm, obu_hbm, obi_hbm, obj_hbm,
          u_v, i_v, j_v, vu, vi, vj, bu_v, bi_v, bj_v, drain_v, sem):
        wid = lax.axis_index("s") * NC + lax.axis_index("c")
        base = wid * BPW

        pltpu.sync_copy(u_hbm.at[wid], u_v)
        pltpu.sync_copy(i_hbm.at[wid], i_v)
        pltpu.sync_copy(j_hbm.at[wid], j_v)

        bias_copies = []
        for t in range(NCHUNK):
            sl = pl.ds(t * CHUNK, CHUNK)
            bias_copies.append(pltpu.async_copy(
                sbu_hbm.at[u_v.at[t]], bu_v.at[sl], sem))
            bias_copies.append(pltpu.async_copy(
                sbi_hbm.at[i_v.at[t]], bi_v.at[sl], sem))
            bias_copies.append(pltpu.async_copy(
                sbi_hbm.at[j_v.at[t]], bj_v.at[sl], sem))

        def fire(c, carry):
            for t in range(NCHUNK):
                sl = pl.ds(t * CHUNK, CHUNK)
                pltpu.async_copy(su_hbm.at[c].at[u_v.at[t]], vu.at[c, sl], sem)
                pltpu.async_copy(si_hbm.at[c].at[i_v.at[t]], vi.at[c, sl], sem)
                pltpu.async_copy(si_hbm.at[c].at[j_v.at[t]], vj.at[c, sl], sem)

            @pl.when(c > 1)
            def _():
                pltpu.make_async_copy(
                    sbu_hbm.at[pl.ds(0, 3 * NCHUNK * CHUNK)], drain_v, sem).wait()
            return carry

        lax.fori_loop(0, K, fire, 0)
        pltpu.make_async_copy(
            sbu_hbm.at[pl.ds(0, 3 * NCHUNK * CHUNK)], drain_v, sem).wait()
        pltpu.make_async_copy(
            sbu_hbm.at[pl.ds(0, 3 * NCHUNK * CHUNK)], drain_v, sem).wait()
        for cp in bias_copies:
            cp.wait()

        out_copies = [
            pltpu.async_copy(bu_v, obu_hbm.at[pl.ds(base, BPW)], sem),
            pltpu.async_copy(bi_v, obi_hbm.at[pl.ds(base, BPW)], sem),
            pltpu.async_copy(bj_v, obj_hbm.at[pl.ds(base, BPW)], sem),
        ]
        for c in range(K):
            out_copies.append(pltpu.async_copy(
                vu.at[c], lu_hbm.at[pl.ds(c * B + base, BPW)], sem))
            out_copies.append(pltpu.async_copy(
                vi.at[c], li_hbm.at[pl.ds(c * B + base, BPW)], sem))
            out_copies.append(pltpu.async_copy(
                vj.at[c], lj_hbm.at[pl.ds(c * B + base, BPW)], sem))
        for cp in out_copies:
            cp.wait()

    return k


_bpr_stage = _stage_kernel()
_bpr_gather = _gather_kernel()


def _epilogue_body(u_ref, i_ref, j_ref, lu_ref, li_ref, lj_ref,
                   bu_ref, bi_ref, bj_ref,
                   tgu_ref, tgi_ref, tbu_ref, tbi_ref,
                   xui_ref, xuj_ref, loss_ref):
    u = u_ref[...]
    i = i_ref[...]
    j = j_ref[...]
    tail_ids = lax.broadcasted_iota(jnp.int32, (NTAIL, B), 0) + T0

    def fix(latent, bias_vals, idx, tg, tb):
        onehot = (tail_ids == idx).astype(jnp.float32)
        is_tail = idx >= T0
        lat_fix = jnp.dot(tg, onehot,
                          preferred_element_type=jnp.float32)
        b_fix = jnp.dot(tb, onehot,
                        preferred_element_type=jnp.float32)
        latent = jnp.where(is_tail, lat_fix, latent)
        bias_vals = jnp.where(is_tail, b_fix, bias_vals)
        return latent, bias_vals

    lu, bu = fix(lu_ref[...], bu_ref[...], u, tgu_ref[...], tbu_ref[...])
    li, bi = fix(li_ref[...], bi_ref[...], i, tgi_ref[...], tbi_ref[...])
    lj, bj = fix(lj_ref[...], bj_ref[...], j, tgi_ref[...], tbi_ref[...])

    x_ui = jnp.sum(lu * li, axis=0, keepdims=True) + bu + bi
    x_uj = jnp.sum(lu * lj, axis=0, keepdims=True) + bu + bj
    xui_ref[...] = x_ui
    xuj_ref[...] = x_uj
    d = x_ui - x_uj
    ls = jnp.minimum(d, 0.0) - jnp.log(1.0 + jnp.exp(-jnp.abs(d)))
    loss_ref[...] = jnp.broadcast_to(-jnp.sum(ls) / B, (1, 1))


def _epilogue_tc(u, i, j, lu, li, lj, bu, bi, bj, tgu, tgi, tbu, tbi):
    return pl.pallas_call(
        _epilogue_body,
        out_shape=[
            jax.ShapeDtypeStruct((1, B), jnp.float32),
            jax.ShapeDtypeStruct((1, B), jnp.float32),
            jax.ShapeDtypeStruct((1, 1), jnp.float32),
        ],
    )(u.reshape(1, B), i.reshape(1, B), j.reshape(1, B),
      lu.reshape(K, B), li.reshape(K, B), lj.reshape(K, B),
      bu.reshape(1, B), bi.reshape(1, B), bj.reshape(1, B),
      tgu, tgi, tbu, tbi)


@jax.jit
def kernel(u, i, j, gamma_u, gamma_i, beta_u, beta_i):
    u32 = u.astype(jnp.int32)
    i32 = i.astype(jnp.int32)
    j32 = j.astype(jnp.int32)
    u_r = u32.reshape(NW, NCHUNK, CHUNK)
    i_r = i32.reshape(NW, NCHUNK, CHUNK)
    j_r = j32.reshape(NW, NCHUNK, CHUNK)
    su, si, sbu, sbi = _bpr_stage(gamma_u.T, gamma_i.T, beta_u.T, beta_i.T)
    lu, li, lj, bu, bi, bj = _bpr_gather(u_r, i_r, j_r,
                                         su.reshape(K, N), si.reshape(K, N),
                                         sbu, sbi)
    tgu = gamma_u[T0:].T
    tgi = gamma_i[T0:].T
    tbu = beta_u[T0:].reshape(1, NTAIL)
    tbi = beta_i[T0:].reshape(1, NTAIL)
    x_ui, x_uj, loss = _epilogue_tc(u32, i32, j32, lu, li, lj, bu, bi, bj,
                                    tgu, tgi, tbu, tbi)
    return (x_ui.reshape(B), x_uj.reshape(B), loss[0, 0])


# speedup vs baseline: 3.0667x; 1.0065x over previous
import functools

import jax
import jax.numpy as jnp
from jax import lax
from jax.experimental import pallas as pl
from jax.experimental.pallas import tpu as pltpu
from jax.experimental.pallas import tpu_sc as plsc

K = 32
B = 16384
N = 1000000
NC = 2
NS = 16
NW = NC * NS
BPW = B // NW
CHUNK = 128
NCHUNK = BPW // CHUNK

CB = 512
T0 = (N // 128) * 128
NFULL = T0 // CB
NTAIL = N - T0


def _stage_kernel():
    mesh = plsc.VectorSubcoreMesh(core_axis_name="c", subcore_axis_name="s")

    @functools.partial(
        pl.kernel,
        mesh=mesh,
        out_type=[
            jax.ShapeDtypeStruct((K * N,), jnp.float32),
            jax.ShapeDtypeStruct((K * N,), jnp.float32),
            jax.ShapeDtypeStruct((N,), jnp.float32),
            jax.ShapeDtypeStruct((N,), jnp.float32),
        ],
        scratch_types=[
            pltpu.VMEM((3 * K * CB,), jnp.float32),
            pltpu.VMEM((3 * K * CB,), jnp.float32),
            pltpu.VMEM((3 * CB,), jnp.float32),
            pltpu.VMEM((3 * CB,), jnp.float32),
            pltpu.SemaphoreType.DMA,
            pltpu.SemaphoreType.DMA,
        ],
    )
    def k(guT_hbm, giT_hbm, buT_hbm, biT_hbm,
          su_hbm, si_hbm, sbu_hbm, sbi_hbm,
          bu, bi, bbu, bbi, sem_r, sem_w):
        wid = lax.axis_index("s") * NC + lax.axis_index("c")
        nblk = NFULL // NW + 1

        def fire_reads(par, off):
            pltpu.async_copy(buT_hbm.at[0, pl.ds(off, CB)], bbu.at[pl.ds(par * CB, CB)], sem_r)
            pltpu.async_copy(biT_hbm.at[0, pl.ds(off, CB)], bbi.at[pl.ds(par * CB, CB)], sem_r)
            for c in range(K):
                csl = pl.ds(c * CB, CB)
                pltpu.async_copy(
                    guT_hbm.at[c, pl.ds(off, CB)], bu.at[pl.ds(par * K * CB + c * CB, CB)], sem_r)
                pltpu.async_copy(
                    giT_hbm.at[c, pl.ds(off, CB)], bi.at[pl.ds(par * K * CB + c * CB, CB)], sem_r)

        def fire_writes(par, off):
            pltpu.async_copy(bbu.at[pl.ds(par * CB, CB)], sbu_hbm.at[pl.ds(off, CB)], sem_w)
            pltpu.async_copy(bbi.at[pl.ds(par * CB, CB)], sbi_hbm.at[pl.ds(off, CB)], sem_w)
            for c in range(K):
                csl = pl.ds(c * CB, CB)
                pltpu.async_copy(
                    bu.at[pl.ds(par * K * CB + c * CB, CB)], su_hbm.at[pl.ds(c * N + off, CB)], sem_w)
                pltpu.async_copy(
                    bi.at[pl.ds(par * K * CB + c * CB, CB)], si_hbm.at[pl.ds(c * N + off, CB)], sem_w)

        def drain(sem):
            pltpu.make_async_copy(
                buT_hbm.at[0, pl.ds(0, CB)], bbu.at[pl.ds(0, CB)], sem).wait()
            pltpu.make_async_copy(
                buT_hbm.at[0, pl.ds(0, CB)], bbi.at[pl.ds(0, CB)], sem).wait()
            for c in range(K):
                csl = pl.ds(c * CB, CB)
                pltpu.make_async_copy(
                    guT_hbm.at[0, pl.ds(0, CB)], bu.at[csl], sem).wait()
                pltpu.make_async_copy(
                    guT_hbm.at[0, pl.ds(0, CB)], bi.at[csl], sem).wait()

        def blk_off(n):
            return pl.multiple_of((wid + n * NW) * CB, CB)

        fire_reads(0, blk_off(0))

        def body(n, carry):
            blk = wid + n * NW

            @pl.when(blk < NFULL)
            def _():
                @pl.when(n > 1)
                def _():
                    drain(sem_w)

                @pl.when(wid + (n + 1) * NW < NFULL)
                def _():
                    for par in (0, 1, 2):
                        @pl.when((n + 1) % 3 == par)
                        def _():
                            fire_reads(par, blk_off(n + 1))

                drain(sem_r)
                for par in (0, 1, 2):
                    @pl.when(n % 3 == par)
                    def _():
                        fire_writes(par, blk_off(n))
            return carry

        lax.fori_loop(0, nblk, body, 0)
        drain(sem_w)
        drain(sem_w)

    return k


def _gather_kernel():
    mesh = plsc.VectorSubcoreMesh(core_axis_name="c", subcore_axis_name="s")

    @functools.partial(
        pl.kernel,
        mesh=mesh,
        compiler_params=pltpu.CompilerParams(use_tc_tiling_on_sc=False),
        out_type=[
            jax.ShapeDtypeStruct((K * B,), jnp.float32),
            jax.ShapeDtypeStruct((K * B,), jnp.float32),
            jax.ShapeDtypeStruct((K * B,), jnp.float32),
            jax.ShapeDtypeStruct((B,), jnp.float32),
            jax.ShapeDtypeStruct((B,), jnp.float32),
            jax.ShapeDtypeStruct((B,), jnp.float32),
        ],
        scratch_types=[
            pltpu.VMEM((NCHUNK, CHUNK), jnp.int32),
            pltpu.VMEM((NCHUNK, CHUNK), jnp.int32),
            pltpu.VMEM((NCHUNK, CHUNK), jnp.int32),
            pltpu.VMEM((K, BPW), jnp.float32),
            pltpu.VMEM((K, BPW), jnp.float32),
            pltpu.VMEM((K, BPW), jnp.float32),
            pltpu.VMEM((BPW,), jnp.float32),
            pltpu.VMEM((BPW,), jnp.float32),
            pltpu.VMEM((BPW,), jnp.float32),
            pltpu.VMEM((3 * NCHUNK * CHUNK,), jnp.float32),
            pltpu.SemaphoreType.DMA,
            pltpu.SemaphoreType.DMA,
        ],
    )
    def k(u_hbm, i_hbm, j_hbm, su_hbm, si_hbm, sbu_hbm, sbi_hbm,
          lu_hbm, li_hbm, lj_hbm, obu_hbm, obi_hbm, obj_hbm,
          u_v, i_v, j_v, vu, vi, vj, bu_v, bi_v, bj_v, drain_v, sem, sem_o):
        wid = lax.axis_index("s") * NC + lax.axis_index("c")
        base = wid * BPW

        pltpu.sync_copy(u_hbm.at[wid], u_v)
        pltpu.sync_copy(i_hbm.at[wid], i_v)
        pltpu.sync_copy(j_hbm.at[wid], j_v)

        bias_copies = []
        for t in range(NCHUNK):
            sl = pl.ds(t * CHUNK, CHUNK)
            bias_copies.append(pltpu.async_copy(
                sbu_hbm.at[u_v.at[t]], bu_v.at[sl], sem))
            bias_copies.append(pltpu.async_copy(
                sbi_hbm.at[i_v.at[t]], bi_v.at[sl], sem))
            bias_copies.append(pltpu.async_copy(
                sbi_hbm.at[j_v.at[t]], bj_v.at[sl], sem))

        def fire(c, carry):
            for t in range(NCHUNK):
                sl = pl.ds(t * CHUNK, CHUNK)
                pltpu.async_copy(su_hbm.at[c].at[u_v.at[t]], vu.at[c, sl], sem)
                pltpu.async_copy(si_hbm.at[c].at[i_v.at[t]], vi.at[c, sl], sem)
                pltpu.async_copy(si_hbm.at[c].at[j_v.at[t]], vj.at[c, sl], sem)

            @pl.when(c > 1)
            def _():
                pltpu.make_async_copy(
                    sbu_hbm.at[pl.ds(0, 3 * NCHUNK * CHUNK)], drain_v, sem).wait()
                c2 = c - 2
                pltpu.async_copy(
                    vu.at[c2], lu_hbm.at[pl.ds(c2 * B + base, BPW)], sem_o)
                pltpu.async_copy(
                    vi.at[c2], li_hbm.at[pl.ds(c2 * B + base, BPW)], sem_o)
                pltpu.async_copy(
                    vj.at[c2], lj_hbm.at[pl.ds(c2 * B + base, BPW)], sem_o)
            return carry

        lax.fori_loop(0, K, fire, 0)
        last_outs = []
        for c in (K - 2, K - 1):
            pltpu.make_async_copy(
                sbu_hbm.at[pl.ds(0, 3 * NCHUNK * CHUNK)], drain_v, sem).wait()
            last_outs.append(pltpu.async_copy(
                vu.at[c], lu_hbm.at[pl.ds(c * B + base, BPW)], sem_o))
            last_outs.append(pltpu.async_copy(
                vi.at[c], li_hbm.at[pl.ds(c * B + base, BPW)], sem_o))
            last_outs.append(pltpu.async_copy(
                vj.at[c], lj_hbm.at[pl.ds(c * B + base, BPW)], sem_o))
        for cp in bias_copies:
            cp.wait()
        out_copies = [
            pltpu.async_copy(bu_v, obu_hbm.at[pl.ds(base, BPW)], sem),
            pltpu.async_copy(bi_v, obi_hbm.at[pl.ds(base, BPW)], sem),
            pltpu.async_copy(bj_v, obj_hbm.at[pl.ds(base, BPW)], sem),
        ]
        for cp in out_copies:
            cp.wait()
        for _ in range(K - 2):
            pltpu.make_async_copy(
                sbu_hbm.at[pl.ds(0, 3 * NCHUNK * CHUNK)], drain_v, sem_o).wait()
        for cp in last_outs:
            cp.wait()

    return k


_bpr_stage = _stage_kernel()
_bpr_gather = _gather_kernel()


def _epilogue_body(u_ref, i_ref, j_ref, lu_ref, li_ref, lj_ref,
                   bu_ref, bi_ref, bj_ref,
                   tgu_ref, tgi_ref, tbu_ref, tbi_ref,
                   xui_ref, xuj_ref, loss_ref):
    u = u_ref[...]
    i = i_ref[...]
    j = j_ref[...]
    tail_ids = lax.broadcasted_iota(jnp.int32, (NTAIL, B), 0) + T0

    def fix(latent, bias_vals, idx, tg, tb):
        onehot = (tail_ids == idx).astype(jnp.float32)
        is_tail = idx >= T0
        lat_fix = jnp.dot(tg, onehot,
                          preferred_element_type=jnp.float32)
        b_fix = jnp.dot(tb, onehot,
                        preferred_element_type=jnp.float32)
        latent = jnp.where(is_tail, lat_fix, latent)
        bias_vals = jnp.where(is_tail, b_fix, bias_vals)
        return latent, bias_vals

    lu, bu = fix(lu_ref[...], bu_ref[...], u, tgu_ref[...], tbu_ref[...])
    li, bi = fix(li_ref[...], bi_ref[...], i, tgi_ref[...], tbi_ref[...])
    lj, bj = fix(lj_ref[...], bj_ref[...], j, tgi_ref[...], tbi_ref[...])

    x_ui = jnp.sum(lu * li, axis=0, keepdims=True) + bu + bi
    x_uj = jnp.sum(lu * lj, axis=0, keepdims=True) + bu + bj
    xui_ref[...] = x_ui
    xuj_ref[...] = x_uj
    d = x_ui - x_uj
    ls = jnp.minimum(d, 0.0) - jnp.log(1.0 + jnp.exp(-jnp.abs(d)))
    loss_ref[...] = jnp.broadcast_to(-jnp.sum(ls) / B, (1, 1))


def _epilogue_tc(u, i, j, lu, li, lj, bu, bi, bj, tgu, tgi, tbu, tbi):
    return pl.pallas_call(
        _epilogue_body,
        out_shape=[
            jax.ShapeDtypeStruct((1, B), jnp.float32),
            jax.ShapeDtypeStruct((1, B), jnp.float32),
            jax.ShapeDtypeStruct((1, 1), jnp.float32),
        ],
    )(u.reshape(1, B), i.reshape(1, B), j.reshape(1, B),
      lu.reshape(K, B), li.reshape(K, B), lj.reshape(K, B),
      bu.reshape(1, B), bi.reshape(1, B), bj.reshape(1, B),
      tgu, tgi, tbu, tbi)


@jax.jit
def kernel(u, i, j, gamma_u, gamma_i, beta_u, beta_i):
    u32 = u.astype(jnp.int32)
    i32 = i.astype(jnp.int32)
    j32 = j.astype(jnp.int32)
    u_r = u32.reshape(NW, NCHUNK, CHUNK)
    i_r = i32.reshape(NW, NCHUNK, CHUNK)
    j_r = j32.reshape(NW, NCHUNK, CHUNK)
    su, si, sbu, sbi = _bpr_stage(gamma_u.T, gamma_i.T, beta_u.T, beta_i.T)
    lu, li, lj, bu, bi, bj = _bpr_gather(u_r, i_r, j_r,
                                         su.reshape(K, N), si.reshape(K, N),
                                         sbu, sbi)
    tgu = gamma_u[T0:].T
    tgi = gamma_i[T0:].T
    tbu = beta_u[T0:].reshape(1, NTAIL)
    tbi = beta_i[T0:].reshape(1, NTAIL)
    x_ui, x_uj, loss = _epilogue_tc(u32, i32, j32, lu, li, lj, bu, bi, bj,
                                    tgu, tgi, tbu, tbi)
    return (x_ui.reshape(B), x_uj.reshape(B), loss[0, 0])
